# baseline (speedup 1.0000x reference)
import functools

import jax
import jax.numpy as jnp
import numpy as np
from jax.experimental import pallas as pl
from jax.experimental.pallas import tpu as pltpu

_TIMESTEPS = 100
_ROWS_PER_BLOCK = 512


def _threefry_bits(e):
    ks0 = jnp.uint32(0)
    ks1 = jnp.uint32(42)
    ks2 = jnp.uint32(0x1BD11BDA ^ 42)
    rot1 = (13, 15, 26, 6)
    rot2 = (17, 29, 16, 24)

    x0 = jnp.zeros_like(e)
    x1 = e + ks1

    def rounds(x0, x1, rots):
        for r in rots:
            x0 = x0 + x1
            x1 = ((x1 << jnp.uint32(r)) | (x1 >> jnp.uint32(32 - r))) ^ x0
        return x0, x1

    x0, x1 = rounds(x0, x1, rot1)
    x0 = x0 + ks1
    x1 = x1 + (ks2 + jnp.uint32(1))
    x0, x1 = rounds(x0, x1, rot2)
    x0 = x0 + ks2
    x1 = x1 + (ks0 + jnp.uint32(2))
    x0, x1 = rounds(x0, x1, rot1)
    x0 = x0 + ks0
    x1 = x1 + (ks1 + jnp.uint32(3))
    x0, x1 = rounds(x0, x1, rot2)
    x0 = x0 + ks1
    x1 = x1 + (ks2 + jnp.uint32(4))
    x0, x1 = rounds(x0, x1, rot1)
    x0 = x0 + ks2
    x1 = x1 + (ks0 + jnp.uint32(5))
    return x0 ^ x1


def _mask_body(m_ref, mask_ref):
    blk = pl.program_id(0)
    r, n = mask_ref.shape
    row = jnp.uint32(blk * r) + jax.lax.broadcasted_iota(jnp.uint32, (r, n), 0)
    col = jax.lax.broadcasted_iota(jnp.uint32, (r, n), 1)
    e = row * jnp.uint32(n) + col
    bits = _threefry_bits(e)
    mask = (bits >> jnp.uint32(9)) < m_ref[...].astype(jnp.uint32)
    mask_ref[...] = mask.astype(jnp.int32)


@functools.partial(jax.jit)
def kernel(x_0, t):
    b, n = x_0.shape
    p = t.astype(jnp.float32) / _TIMESTEPS
    m = jnp.ceil(p.astype(jnp.float64) * (2.0 ** 23)).astype(jnp.int32)
    m = m.reshape(b, 1)

    r = _ROWS_PER_BLOCK
    grid = (b // r,)
    _imap = lambda i: (i, np.int32(0))
    mask01 = pl.pallas_call(
        _mask_body,
        grid=grid,
        in_specs=[pl.BlockSpec((r, 1), _imap)],
        out_specs=pl.BlockSpec((r, n), _imap),
        out_shape=jax.ShapeDtypeStruct((b, n), jnp.int32),
        compiler_params=pltpu.CompilerParams(
            dimension_semantics=("parallel",),
        ),
    )(m)
    mb = mask01 != 0
    x_t = jnp.where(mb, jnp.asarray(21, dtype=x_0.dtype), x_0)
    x_0_ignore = jnp.where(mb, x_0, jnp.asarray(-1, dtype=x_0.dtype))
    return (x_t, x_0_ignore)
